# baseline (speedup 1.0000x reference)
import functools

import jax
import jax.numpy as jnp
from jax import lax
from jax.experimental import pallas as pl
from jax.experimental.pallas import tpu as pltpu
from jax.experimental.pallas import tpu_sc as plsc

NBINS = 2048
SCALE = float(NBINS) * (1.0 - 2.0 ** -20)
LANES = 16
NCORES = 2
NSUB = 16
NWORKERS = NCORES * NSUB
CHUNK = 16384


def _tc_softmax_err_body(lg_ref, lab_ref, z_ref):
    x = lg_ref[0]
    m = jnp.max(x, axis=0, keepdims=True)
    ex = jnp.exp(x - m)
    r = SCALE / jnp.sum(ex, axis=0, keepdims=True)
    x1 = ex * r
    lab = lab_ref[0]
    C, BH, W = x.shape
    cls = lax.broadcasted_iota(jnp.int32, (C, BH, W), 0)
    fg = lab[None, :, :] == cls
    u = jnp.floor(jnp.where(fg, (float(NBINS) + SCALE) - x1, x1))
    z_ref[0] = u.astype(jnp.int16)


def _tc_softmax_err(logits, labels):
    B, C, H, W = logits.shape
    BH = 64
    return pl.pallas_call(
        _tc_softmax_err_body,
        grid=(B, H // BH),
        in_specs=[
            pl.BlockSpec((1, C, BH, W), lambda b, h: (b, 0, h, 0)),
            pl.BlockSpec((1, BH, W), lambda b, h: (b, h, 0)),
        ],
        out_specs=pl.BlockSpec((1, C, BH, W), lambda b, h: (b, 0, h, 0)),
        out_shape=jax.ShapeDtypeStruct((B, C, H, W), jnp.int16),
    )(logits, labels)


def _sc_losses(z4):
    B, C, H, W = z4.shape
    n_rows = B * C
    n_pix = H * W
    hrows_per_chunk = CHUNK // W
    rows_padded = ((n_rows + NWORKERS - 1) // NWORKERS) * NWORKERS
    rows_per_w = rows_padded // NWORKERS
    n_chunks = n_pix // CHUNK
    vec_per_chunk = CHUNK // LANES
    hist_words = LANES * 2 * NBINS
    mesh = plsc.VectorSubcoreMesh(core_axis_name="c", subcore_axis_name="s")

    @functools.partial(
        pl.kernel,
        out_type=jax.ShapeDtypeStruct((rows_padded, LANES), jnp.float32),
        mesh=mesh,
        scratch_types=[
            pltpu.VMEM((hist_words,), jnp.float32),
            pltpu.VMEM((2 * NBINS,), jnp.float32),
            pltpu.VMEM((hrows_per_chunk, W), jnp.int16),
            pltpu.VMEM((hrows_per_chunk, W), jnp.int16),
            pltpu.VMEM((LANES,), jnp.float32),
            pltpu.SemaphoreType.DMA,
            pltpu.SemaphoreType.DMA,
        ],
        compiler_params=pltpu.CompilerParams(
            use_tc_tiling_on_sc=False, needs_layout_passes=False),
    )
    def body(z_hbm, out_hbm, hist, hred, zbuf0, zbuf1, outbuf, sem0, sem1):
        wid = lax.axis_index("s") * NCORES + lax.axis_index("c")
        lane = lax.iota(jnp.int32, LANES)
        ones = jnp.ones((LANES,), jnp.float32)
        zerov = jnp.zeros((LANES,), jnp.float32)

        @plsc.parallel_loop(0, hist_words // LANES, unroll=8)
        def _(i):
            hist[pl.ds(i * LANES, LANES)] = zerov

        for t in range(rows_per_w):
            row = wid + NWORKERS * t
            rb = row // C
            rc = row % C

            @pl.when(row >= n_rows)
            def _():
                outbuf[...] = zerov
                pltpu.sync_copy(outbuf, out_hbm.at[row])

            @pl.when(row < n_rows)
            def _():
                def process(buf):
                    @plsc.parallel_loop(0, hrows_per_chunk, unroll=2)
                    def _(r):
                        for j in range(W // (2 * LANES)):
                            zz = buf[r, pl.ds(j * 2 * LANES, 2 * LANES)]
                            a, b = plsc.unpack(
                                zz, format=plsc.PackFormat.INTERLEAVED,
                                preferred_element_type=jnp.int32)
                            plsc.addupdate_scatter(
                                hist, [a * LANES + lane], ones)
                            plsc.addupdate_scatter(
                                hist, [b * LANES + lane], ones)

                def src(c):
                    return z_hbm.at[rb, rc,
                                    pl.ds(c * hrows_per_chunk,
                                          hrows_per_chunk), :]

                pltpu.async_copy(src(0), zbuf0, sem0)

                def chunk_body(i, carry):
                    c0 = 2 * i
                    pltpu.make_async_copy(src(c0), zbuf0, sem0).wait()
                    pltpu.async_copy(src(c0 + 1), zbuf1, sem1)
                    process(zbuf0)
                    pltpu.make_async_copy(src(c0 + 1), zbuf1, sem1).wait()

                    @pl.when(c0 + 2 < n_chunks)
                    def _():
                        pltpu.async_copy(src(c0 + 2), zbuf0, sem0)
                    process(zbuf1)
                    return carry
                lax.fori_loop(0, n_chunks // 2, chunk_body, 0)

                @plsc.parallel_loop(0, 2 * NBINS // LANES, carry=zerov)
                def gvec(tile, gv):
                    acc = zerov
                    for j in range(LANES):
                        v = hist[pl.ds((tile * LANES + j) * LANES, LANES)]
                        hist[pl.ds((tile * LANES + j) * LANES, LANES)] = zerov
                        acc = jnp.where(lane == j, jnp.sum(v), acc)
                    hred[pl.ds(tile * LANES, LANES)] = acc
                    return gv + jnp.where(tile >= NBINS // LANES, acc, 0.0)
                G = jnp.sum(gvec)

                @plsc.parallel_loop(
                    0, NBINS // LANES,
                    carry=(jnp.float32(0.0), jnp.float32(0.0), zerov))
                def jac_carry(j, carry):
                    cN, cS, jvec = carry
                    jj = NBINS // LANES - 1 - j
                    h0 = hred[pl.ds(jj * LANES, LANES)]
                    h1 = hred[pl.ds(NBINS + jj * LANES, LANES)]
                    rn = lax.rev(h0 + h1, (0,))
                    rs = lax.rev(h1, (0,))
                    cn = plsc.cumsum(rn) + cN
                    cs = plsc.cumsum(rs) + cS
                    union = jnp.maximum(G + cn - cs, 1e-30)
                    J = jnp.where(cn > 0.5, 1.0 - (G - cs) / union, 0.0)
                    return jnp.max(cn), jnp.max(cs), jvec + J
                _, _, jvec = jac_carry

                loss = (jnp.sum(jvec) - 0.5) * (1.0 / NBINS)
                outbuf[...] = zerov + loss
                pltpu.sync_copy(outbuf, out_hbm.at[row])

    return body(z4)


def kernel(logits, labels):
    B, C, H, W = logits.shape
    z = _tc_softmax_err(logits, labels)
    out = _sc_losses(z)
    return jnp.sum(out) * (1.0 / (LANES * B * C))


# speedup vs baseline: 1.0861x; 1.0861x over previous
import functools

import jax
import jax.numpy as jnp
from jax import lax
from jax.experimental import pallas as pl
from jax.experimental.pallas import tpu as pltpu
from jax.experimental.pallas import tpu_sc as plsc

NBINS = 2048
SCALE = float(NBINS) * (1.0 - 2.0 ** -20)
LANES = 16
NCORES = 2
NSUB = 16
NWORKERS = NCORES * NSUB
CHUNK = 16384


def _tc_softmax_err_body(lg_ref, lab_ref, z_ref):
    x = lg_ref[0]
    m = jnp.max(x, axis=0, keepdims=True)
    ex = jnp.exp(x - m)
    r = SCALE / jnp.sum(ex, axis=0, keepdims=True)
    x1 = ex * r
    lab = lab_ref[0]
    C, BH, W = x.shape
    cls = lax.broadcasted_iota(jnp.int32, (C, BH, W), 0)
    fg = lab[None, :, :] == cls
    u = jnp.floor(jnp.where(fg, (float(NBINS) + SCALE) - x1, x1))
    wlane = lax.broadcasted_iota(jnp.int32, (C, BH, W), 2) & (LANES - 1)
    z_ref[0] = u * 16.0 + wlane.astype(jnp.float32)


def _tc_softmax_err(logits, labels):
    B, C, H, W = logits.shape
    BH = 64
    return pl.pallas_call(
        _tc_softmax_err_body,
        grid=(B, H // BH),
        in_specs=[
            pl.BlockSpec((1, C, BH, W), lambda b, h: (b, 0, h, 0)),
            pl.BlockSpec((1, BH, W), lambda b, h: (b, h, 0)),
        ],
        out_specs=pl.BlockSpec((1, C, BH, W), lambda b, h: (b, 0, h, 0)),
        out_shape=jax.ShapeDtypeStruct((B, C, H, W), jnp.float32),
    )(logits, labels)


def _sc_losses(z4):
    B, C, H, W = z4.shape
    n_rows = B * C
    n_pix = H * W
    hrows_per_chunk = CHUNK // W
    rows_padded = ((n_rows + NWORKERS - 1) // NWORKERS) * NWORKERS
    rows_per_w = rows_padded // NWORKERS
    n_chunks = n_pix // CHUNK
    vec_per_chunk = CHUNK // LANES
    hist_words = LANES * 2 * NBINS
    mesh = plsc.VectorSubcoreMesh(core_axis_name="c", subcore_axis_name="s")

    @functools.partial(
        pl.kernel,
        out_type=jax.ShapeDtypeStruct((rows_padded, LANES), jnp.float32),
        mesh=mesh,
        scratch_types=[
            pltpu.VMEM((hist_words,), jnp.float32),
            pltpu.VMEM((2 * NBINS,), jnp.float32),
            pltpu.VMEM((hrows_per_chunk, W), jnp.float32),
            pltpu.VMEM((hrows_per_chunk, W), jnp.float32),
            pltpu.VMEM((LANES,), jnp.float32),
            pltpu.SemaphoreType.DMA,
            pltpu.SemaphoreType.DMA,
        ],
        compiler_params=pltpu.CompilerParams(
            use_tc_tiling_on_sc=False, needs_layout_passes=False),
    )
    def body(z_hbm, out_hbm, hist, hred, zbuf0, zbuf1, outbuf, sem0, sem1):
        wid = lax.axis_index("s") * NCORES + lax.axis_index("c")
        lane = lax.iota(jnp.int32, LANES)
        ones = jnp.ones((LANES,), jnp.float32)
        zerov = jnp.zeros((LANES,), jnp.float32)

        @plsc.parallel_loop(0, hist_words // LANES, unroll=8)
        def _(i):
            hist[pl.ds(i * LANES, LANES)] = zerov

        for t in range(rows_per_w):
            row = wid + NWORKERS * t
            rb = row // C
            rc = row % C

            @pl.when(row >= n_rows)
            def _():
                outbuf[...] = zerov
                pltpu.sync_copy(outbuf, out_hbm.at[row])

            @pl.when(row < n_rows)
            def _():
                def process(buf):
                    @plsc.parallel_loop(0, hrows_per_chunk, unroll=2)
                    def _(r):
                        for j in range(W // LANES):
                            zz = buf[r, pl.ds(j * LANES, LANES)]
                            plsc.addupdate_scatter(
                                hist, [zz.astype(jnp.int32)], ones)

                def src(c):
                    return z_hbm.at[rb, rc,
                                    pl.ds(c * hrows_per_chunk,
                                          hrows_per_chunk), :]

                pltpu.async_copy(src(0), zbuf0, sem0)

                def chunk_body(i, carry):
                    c0 = 2 * i
                    pltpu.make_async_copy(src(c0), zbuf0, sem0).wait()
                    pltpu.async_copy(src(c0 + 1), zbuf1, sem1)
                    process(zbuf0)
                    pltpu.make_async_copy(src(c0 + 1), zbuf1, sem1).wait()

                    @pl.when(c0 + 2 < n_chunks)
                    def _():
                        pltpu.async_copy(src(c0 + 2), zbuf0, sem0)
                    process(zbuf1)
                    return carry
                lax.fori_loop(0, n_chunks // 2, chunk_body, 0)

                @plsc.parallel_loop(0, 2 * NBINS // LANES, carry=zerov)
                def gvec(tile, gv):
                    acc = zerov
                    for j in range(LANES):
                        v = hist[pl.ds((tile * LANES + j) * LANES, LANES)]
                        hist[pl.ds((tile * LANES + j) * LANES, LANES)] = zerov
                        acc = jnp.where(lane == j, jnp.sum(v), acc)
                    hred[pl.ds(tile * LANES, LANES)] = acc
                    return gv + jnp.where(tile >= NBINS // LANES, acc, 0.0)
                G = jnp.sum(gvec)

                @plsc.parallel_loop(
                    0, NBINS // LANES,
                    carry=(jnp.float32(0.0), jnp.float32(0.0), zerov))
                def jac_carry(j, carry):
                    cN, cS, jvec = carry
                    jj = NBINS // LANES - 1 - j
                    h0 = hred[pl.ds(jj * LANES, LANES)]
                    h1 = hred[pl.ds(NBINS + jj * LANES, LANES)]
                    rn = lax.rev(h0 + h1, (0,))
                    rs = lax.rev(h1, (0,))
                    cn = plsc.cumsum(rn) + cN
                    cs = plsc.cumsum(rs) + cS
                    union = jnp.maximum(G + cn - cs, 1e-30)
                    J = jnp.where(cn > 0.5, 1.0 - (G - cs) / union, 0.0)
                    return jnp.max(cn), jnp.max(cs), jvec + J
                _, _, jvec = jac_carry

                loss = (jnp.sum(jvec) - 0.5) * (1.0 / NBINS)
                outbuf[...] = zerov + loss
                pltpu.sync_copy(outbuf, out_hbm.at[row])

    return body(z4)


def kernel(logits, labels):
    B, C, H, W = logits.shape
    z = _tc_softmax_err(logits, labels)
    out = _sc_losses(z)
    return jnp.sum(out) * (1.0 / (LANES * B * C))
